# Initial kernel scaffold; baseline (speedup 1.0000x reference)
import functools

import jax
import jax.numpy as jnp
from jax import lax
from jax.experimental import pallas as pl
from jax.experimental.pallas import tpu as pltpu
from jax.experimental.pallas import tpu_sc as plsc

VEC = 16
NSUB = 16
NCORE = 2
BLK = 128
F32 = jnp.float32
I32 = jnp.int32


def _ceil_to(x, m):
    return (x + m - 1) // m * m


def _make_prep(EP, N, R, CNT_P):
    EPW = EP // (NCORE * NSUB)
    NBLK = EPW // BLK
    CPT = CNT_P // NSUB
    NZB = CPT // BLK
    mesh = plsc.VectorSubcoreMesh(core_axis_name="c", subcore_axis_name="s")

    @functools.partial(
        pl.kernel,
        out_type=(
            jax.ShapeDtypeStruct((NCORE, CNT_P), F32),
            jax.ShapeDtypeStruct((EP,), I32),
            jax.ShapeDtypeStruct((NCORE, EP), I32),
        ),
        mesh=mesh,
        scratch_types=[
            pltpu.VMEM((BLK,), I32),
            pltpu.VMEM((BLK,), I32),
            pltpu.VMEM((BLK,), I32),
            pltpu.VMEM((BLK,), I32),
            pltpu.VMEM((BLK,), I32),
            pltpu.VMEM((BLK,), F32),
            pltpu.VMEM_SHARED((CNT_P,), F32),
        ],
    )
    def prep(src_h, dst_h, et_h, cnt_h, key_h, gid_h,
             src_v, dst_v, et_v, key_v, gid_v, one_v, cnt_sp):
        c = lax.axis_index("c")
        s = lax.axis_index("s")
        wid = c * NSUB + s

        @pl.loop(0, BLK // VEC)
        def _(j):
            one_v[pl.ds(j * VEC, VEC)] = jnp.zeros((VEC,), F32)

        @pl.loop(0, NZB)
        def _(k):
            pltpu.sync_copy(one_v, cnt_sp.at[pl.ds(s * CPT + k * BLK, BLK)])

        @pl.loop(0, BLK // VEC)
        def _(j):
            one_v[pl.ds(j * VEC, VEC)] = jnp.ones((VEC,), F32)

        plsc.subcore_barrier()

        @pl.loop(0, NBLK)
        def _(b):
            e0 = wid * EPW + b * BLK
            pltpu.sync_copy(src_h.at[pl.ds(e0, BLK)], src_v)
            pltpu.sync_copy(dst_h.at[pl.ds(e0, BLK)], dst_v)
            pltpu.sync_copy(et_h.at[pl.ds(e0, BLK)], et_v)

            @pl.loop(0, BLK // VEC)
            def _(j):
                sl = pl.ds(j * VEC, VEC)
                key_v[sl] = et_v[sl] * N + dst_v[sl]
                gid_v[sl] = src_v[sl] * 2

            pltpu.sync_copy(key_v, key_h.at[pl.ds(e0, BLK)])
            pltpu.sync_copy(gid_v, gid_h.at[0, pl.ds(e0, BLK)])

            @pl.loop(0, BLK // VEC)
            def _(j):
                sl = pl.ds(j * VEC, VEC)
                gid_v[sl] = gid_v[sl] + 1

            pltpu.sync_copy(gid_v, gid_h.at[1, pl.ds(e0, BLK)])
            pltpu.sync_copy(one_v, cnt_sp.at[key_v], add=True)

        plsc.subcore_barrier()

        @pl.loop(0, NZB)
        def _(k):
            r0 = s * CPT + k * BLK
            pltpu.sync_copy(cnt_sp.at[pl.ds(r0, BLK)], cnt_h.at[c, pl.ds(r0, BLK)])

    return prep


def _make_layer_scatter(EP, CNT_P, HD):
    EPT = EP // NSUB
    NBLK = EPT // BLK
    CPT = CNT_P // NSUB
    NZB = CPT // BLK
    mesh = plsc.VectorSubcoreMesh(core_axis_name="c", subcore_axis_name="s")

    @functools.partial(
        pl.kernel,
        out_type=jax.ShapeDtypeStruct((NCORE, CNT_P, HD), F32),
        mesh=mesh,
        scratch_types=[
            pltpu.VMEM((BLK,), I32),
            pltpu.VMEM((BLK,), I32),
            pltpu.VMEM((BLK, HD), F32),
            pltpu.VMEM((BLK, HD), F32),
            pltpu.VMEM((BLK, HD), F32),
            pltpu.VMEM_SHARED((CNT_P, HD), F32),
        ],
    )
    def layer_scatter(xr_h, ew_h, gid_h, key_h, out_h,
                      gid_v, key_v, x_v, ew_v, msg_v, acc):
        c = lax.axis_index("c")
        s = lax.axis_index("s")

        @pl.loop(0, BLK)
        def _(i):
            for j in range(HD // VEC):
                msg_v[i, pl.ds(j * VEC, VEC)] = jnp.zeros((VEC,), F32)

        @pl.loop(0, NZB)
        def _(k):
            pltpu.sync_copy(msg_v, acc.at[pl.ds(s * CPT + k * BLK, BLK)])

        plsc.subcore_barrier()

        @pl.loop(0, NBLK)
        def _(b):
            e0 = s * EPT + b * BLK
            pltpu.sync_copy(gid_h.at[c, pl.ds(e0, BLK)], gid_v)
            pltpu.sync_copy(key_h.at[pl.ds(e0, BLK)], key_v)
            pltpu.sync_copy(ew_h.at[c, pl.ds(e0, BLK)], ew_v)
            pltpu.sync_copy(xr_h.at[gid_v], x_v)

            @pl.loop(0, BLK)
            def _(i):
                for j in range(HD // VEC):
                    sl = pl.ds(j * VEC, VEC)
                    msg_v[i, sl] = jnp.maximum(x_v[i, sl] * ew_v[i, sl], 0.0)

            pltpu.sync_copy(msg_v, acc.at[key_v], add=True)

        plsc.subcore_barrier()

        @pl.loop(0, NZB)
        def _(k):
            r0 = s * CPT + k * BLK
            pltpu.sync_copy(acc.at[pl.ds(r0, BLK)], out_h.at[c, pl.ds(r0, BLK)])

    return layer_scatter


def _mlp_body(ea_ref, w_ref, b_ref, out_ref):
    ew = lax.dot_general(ea_ref[...], w_ref[...], (((1,), (1,)), ((), ())),
                         preferred_element_type=F32,
                         precision=lax.Precision.HIGHEST)
    ew = ew + b_ref[...]
    hd = out_ref.shape[2]
    out_ref[0] = ew[:, :hd]
    out_ref[1] = ew[:, hd:]


def _update_body(x_ref, root_ref, bias_ref, wcat_ref,
                 a00, a01, a02, a10, a11, a12,
                 c00, c01, c02, c10, c11, c12, out_ref):
    xb = x_ref[...]
    out = lax.dot_general(xb, root_ref[...], (((1,), (0,)), ((), ())),
                          preferred_element_type=F32,
                          precision=lax.Precision.HIGHEST)
    out = out + bias_ref[...]
    accs = ((a00, a01, a02), (a10, a11, a12))
    cnts = ((c00, c01, c02), (c10, c11, c12))
    for r in range(3):
        cnt = jnp.maximum(cnts[0][r][0] + cnts[1][r][0], 1.0)
        for c in range(2):
            h = accs[c][r][0] / cnt
            out = out + lax.dot_general(
                h, wcat_ref[c, r], (((1,), (0,)), ((), ())),
                preferred_element_type=F32,
                precision=lax.Precision.HIGHEST)
    out_ref[...] = xb + jnp.maximum(out, 0.0)


def kernel(x, edge_index, edge_attrs, edge_nn_W, edge_nn_b, weights, roots,
           biases):
    N, D = x.shape
    E = edge_index.shape[1]
    L, R = weights.shape[0], weights.shape[1]
    HD = D // 2

    EP = _ceil_to(E, NCORE * NSUB * BLK)
    CNT_P = _ceil_to(R * N + 1, NSUB * BLK)

    src = edge_index[0].astype(I32)
    dst = edge_index[1].astype(I32)
    et = edge_attrs[:, 0].astype(I32)
    ea = edge_attrs[:, 1:]

    pad = EP - E
    src_p = jnp.pad(src, (0, pad))
    dst_p = jnp.pad(dst, (0, pad))
    et_p = jnp.pad(et, (0, pad), constant_values=R)
    ea_p = jnp.pad(ea, ((0, pad), (0, 0)))

    cnt_part, keys, gids = _make_prep(EP, N, R, CNT_P)(src_p, dst_p, et_p)
    cnt3 = cnt_part.reshape(NCORE, CNT_P, 1)

    BE = 512
    ew2 = pl.pallas_call(
        _mlp_body,
        grid=(EP // BE,),
        in_specs=[
            pl.BlockSpec((BE, ea_p.shape[1]), lambda i: (i, 0)),
            pl.BlockSpec(edge_nn_W.shape, lambda i: (0, 0)),
            pl.BlockSpec((1, D), lambda i: (0, 0)),
        ],
        out_specs=pl.BlockSpec((NCORE, BE, HD), lambda i: (0, i, 0)),
        out_shape=jax.ShapeDtypeStruct((NCORE, EP, HD), F32),
    )(ea_p, edge_nn_W, edge_nn_b.reshape(1, D))

    wcat = jnp.transpose(weights.reshape(L, R, NCORE, HD, D), (0, 2, 1, 3, 4))

    layer_scatter = _make_layer_scatter(EP, CNT_P, HD)

    BM = 400
    NB = N // BM
    acc_specs = [
        pl.BlockSpec((1, BM, HD), lambda i, c=c, r=r: (c, r * NB + i, 0))
        for c in range(NCORE) for r in range(R)
    ]
    cnt_specs = [
        pl.BlockSpec((1, BM, 1), lambda i, c=c, r=r: (c, r * NB + i, 0))
        for c in range(NCORE) for r in range(R)
    ]
    update_call = pl.pallas_call(
        _update_body,
        grid=(NB,),
        in_specs=[
            pl.BlockSpec((BM, D), lambda i: (i, 0)),
            pl.BlockSpec((D, D), lambda i: (0, 0)),
            pl.BlockSpec((1, D), lambda i: (0, 0)),
            pl.BlockSpec((NCORE, R, HD, D), lambda i: (0, 0, 0, 0)),
        ] + acc_specs + cnt_specs,
        out_specs=pl.BlockSpec((BM, D), lambda i: (i, 0)),
        out_shape=jax.ShapeDtypeStruct((N, D), F32),
    )

    xcur = x
    for l in range(L):
        xr = xcur.reshape(2 * N, HD)
        acc = layer_scatter(xr, ew2, gids, keys)
        xcur = update_call(xcur, roots[l], biases[l].reshape(1, D), wcat[l],
                           acc, acc, acc, acc, acc, acc,
                           cnt3, cnt3, cnt3, cnt3, cnt3, cnt3)
    return xcur


# speedup vs baseline: 2.2983x; 2.2983x over previous
import functools

import jax
import jax.numpy as jnp
from jax import lax
from jax.experimental import pallas as pl
from jax.experimental.pallas import tpu as pltpu
from jax.experimental.pallas import tpu_sc as plsc

VEC = 16
NSUB = 16
NCORE = 2
PBLK = 128
EBLK = 64
ZB = 40
F32 = jnp.float32
I32 = jnp.int32


def _ceil_to(x, m):
    return (x + m - 1) // m * m


def _make_prep(EP, N, R, CNT_P):
    EPW = EP // (NCORE * NSUB)
    NBLK = EPW // PBLK
    CPT = CNT_P // NSUB
    NZB = CPT // ZB
    mesh = plsc.VectorSubcoreMesh(core_axis_name="c", subcore_axis_name="s")

    @functools.partial(
        pl.kernel,
        out_type=(
            jax.ShapeDtypeStruct((NCORE, CNT_P), F32),
            jax.ShapeDtypeStruct((EP,), I32),
            jax.ShapeDtypeStruct((NCORE, EP), I32),
        ),
        mesh=mesh,
        scratch_types=[
            pltpu.VMEM((PBLK,), I32),
            pltpu.VMEM((PBLK,), I32),
            pltpu.VMEM((PBLK,), I32),
            pltpu.VMEM((PBLK,), I32),
            pltpu.VMEM((PBLK,), I32),
            pltpu.VMEM((PBLK,), F32),
            pltpu.VMEM_SHARED((CNT_P,), F32),
        ],
        compiler_params=pltpu.CompilerParams(use_tc_tiling_on_sc=False),
    )
    def prep(src_h, dst_h, et_h, cnt_h, key_h, gid_h,
             src_v, dst_v, et_v, key_v, gid_v, one_v, cnt_sp):
        c = lax.axis_index("c")
        s = lax.axis_index("s")
        wid = c * NSUB + s

        @pl.loop(0, PBLK // VEC)
        def _(j):
            one_v[pl.ds(j * VEC, VEC)] = jnp.zeros((VEC,), F32)

        @pl.loop(0, NZB)
        def _(k):
            pltpu.sync_copy(one_v.at[pl.ds(0, ZB)],
                            cnt_sp.at[pl.ds(s * CPT + k * ZB, ZB)])

        @pl.loop(0, PBLK // VEC)
        def _(j):
            one_v[pl.ds(j * VEC, VEC)] = jnp.ones((VEC,), F32)

        plsc.subcore_barrier()

        @pl.loop(0, NBLK)
        def _(b):
            e0 = wid * EPW + b * PBLK
            pltpu.sync_copy(src_h.at[pl.ds(e0, PBLK)], src_v)
            pltpu.sync_copy(dst_h.at[pl.ds(e0, PBLK)], dst_v)
            pltpu.sync_copy(et_h.at[pl.ds(e0, PBLK)], et_v)

            @pl.loop(0, PBLK // VEC)
            def _(j):
                sl = pl.ds(j * VEC, VEC)
                key_v[sl] = et_v[sl] * N + dst_v[sl]
                gid_v[sl] = src_v[sl] * 2

            pltpu.sync_copy(key_v, key_h.at[pl.ds(e0, PBLK)])
            pltpu.sync_copy(gid_v, gid_h.at[0, pl.ds(e0, PBLK)])

            @pl.loop(0, PBLK // VEC)
            def _(j):
                sl = pl.ds(j * VEC, VEC)
                gid_v[sl] = gid_v[sl] + 1

            pltpu.sync_copy(gid_v, gid_h.at[1, pl.ds(e0, PBLK)])
            pltpu.sync_copy(one_v, cnt_sp.at[key_v], add=True)

        plsc.subcore_barrier()

        @pl.loop(0, NZB)
        def _(k):
            r0 = s * CPT + k * ZB
            pltpu.sync_copy(cnt_sp.at[pl.ds(r0, ZB)], cnt_h.at[c, pl.ds(r0, ZB)])

    return prep


def _make_layer_scatter(EP, CNT_P, HD):
    EPT = EP // NSUB
    NBLK = EPT // EBLK
    CPT = CNT_P // NSUB
    NZB = CPT // ZB
    mesh = plsc.VectorSubcoreMesh(core_axis_name="c", subcore_axis_name="s")

    @functools.partial(
        pl.kernel,
        out_type=jax.ShapeDtypeStruct((NCORE, CNT_P, HD), F32),
        mesh=mesh,
        scratch_types=[
            pltpu.VMEM((EBLK,), I32),
            pltpu.VMEM((EBLK,), I32),
            pltpu.VMEM((EBLK, HD), F32),
            pltpu.VMEM((EBLK, HD), F32),
            pltpu.VMEM_SHARED((CNT_P, HD), F32),
        ],
        compiler_params=pltpu.CompilerParams(use_tc_tiling_on_sc=False),
    )
    def layer_scatter(xr_h, ew_h, gid_h, key_h, out_h,
                      gid_v, key_v, x_v, ew_v, acc):
        c = lax.axis_index("c")
        s = lax.axis_index("s")

        @pl.loop(0, EBLK)
        def _(i):
            for j in range(HD // VEC):
                ew_v[i, pl.ds(j * VEC, VEC)] = jnp.zeros((VEC,), F32)

        @pl.loop(0, NZB)
        def _(k):
            pltpu.sync_copy(ew_v.at[pl.ds(0, ZB)],
                            acc.at[pl.ds(s * CPT + k * ZB, ZB)])

        plsc.subcore_barrier()

        @pl.loop(0, NBLK)
        def _(b):
            e0 = s * EPT + b * EBLK
            pltpu.sync_copy(gid_h.at[c, pl.ds(e0, EBLK)], gid_v)
            pltpu.sync_copy(key_h.at[pl.ds(e0, EBLK)], key_v)
            pltpu.sync_copy(ew_h.at[c, pl.ds(e0, EBLK)], ew_v)
            pltpu.sync_copy(xr_h.at[gid_v], x_v)

            @pl.loop(0, EBLK)
            def _(i):
                for j in range(HD // VEC):
                    sl = pl.ds(j * VEC, VEC)
                    x_v[i, sl] = jnp.maximum(x_v[i, sl] * ew_v[i, sl], 0.0)

            pltpu.sync_copy(x_v, acc.at[key_v], add=True)

        plsc.subcore_barrier()

        @pl.loop(0, NZB)
        def _(k):
            r0 = s * CPT + k * ZB
            pltpu.sync_copy(acc.at[pl.ds(r0, ZB)], out_h.at[c, pl.ds(r0, ZB)])

    return layer_scatter


def _mlp_body(ea_ref, w_ref, b_ref, out_ref):
    ew = lax.dot_general(ea_ref[...], w_ref[...], (((1,), (1,)), ((), ())),
                         preferred_element_type=F32,
                         precision=lax.Precision.HIGHEST)
    ew = ew + b_ref[...]
    hd = out_ref.shape[2]
    out_ref[0] = ew[:, :hd]
    out_ref[1] = ew[:, hd:]


def _update_body(x_ref, root_ref, bias_ref, wcat_ref,
                 a00, a01, a02, a10, a11, a12,
                 c00, c01, c02, c10, c11, c12, out_ref):
    xb = x_ref[...]
    out = lax.dot_general(xb, root_ref[...], (((1,), (0,)), ((), ())),
                          preferred_element_type=F32,
                          precision=lax.Precision.HIGHEST)
    out = out + bias_ref[...]
    accs = ((a00, a01, a02), (a10, a11, a12))
    cnts = ((c00, c01, c02), (c10, c11, c12))
    for r in range(3):
        cnt = jnp.maximum(cnts[0][r][0] + cnts[1][r][0], 1.0)
        for c in range(2):
            h = accs[c][r][0] / cnt
            out = out + lax.dot_general(
                h, wcat_ref[c, r], (((1,), (0,)), ((), ())),
                preferred_element_type=F32,
                precision=lax.Precision.HIGHEST)
    out_ref[...] = xb + jnp.maximum(out, 0.0)


def kernel(x, edge_index, edge_attrs, edge_nn_W, edge_nn_b, weights, roots,
           biases):
    N, D = x.shape
    E = edge_index.shape[1]
    L, R = weights.shape[0], weights.shape[1]
    HD = D // 2

    EP = _ceil_to(E, NCORE * NSUB * PBLK)
    CNT_P = _ceil_to(R * N + 1, NSUB * ZB)

    src = edge_index[0].astype(I32)
    dst = edge_index[1].astype(I32)
    et = edge_attrs[:, 0].astype(I32)
    ea = edge_attrs[:, 1:]

    pad = EP - E
    src_p = jnp.pad(src, (0, pad))
    dst_p = jnp.pad(dst, (0, pad))
    et_p = jnp.pad(et, (0, pad), constant_values=R)
    ea_p = jnp.pad(ea, ((0, pad), (0, 0)))

    cnt_part, keys, gids = _make_prep(EP, N, R, CNT_P)(src_p, dst_p, et_p)
    cnt3 = cnt_part.reshape(NCORE, CNT_P, 1)

    BE = 512
    ew2 = pl.pallas_call(
        _mlp_body,
        grid=(EP // BE,),
        in_specs=[
            pl.BlockSpec((BE, ea_p.shape[1]), lambda i: (i, 0)),
            pl.BlockSpec(edge_nn_W.shape, lambda i: (0, 0)),
            pl.BlockSpec((1, D), lambda i: (0, 0)),
        ],
        out_specs=pl.BlockSpec((NCORE, BE, HD), lambda i: (0, i, 0)),
        out_shape=jax.ShapeDtypeStruct((NCORE, EP, HD), F32),
    )(ea_p, edge_nn_W, edge_nn_b.reshape(1, D))

    wcat = jnp.transpose(weights.reshape(L, R, NCORE, HD, D), (0, 2, 1, 3, 4))

    layer_scatter = _make_layer_scatter(EP, CNT_P, HD)

    BM = 400
    NB = N // BM
    acc_specs = [
        pl.BlockSpec((1, BM, HD), lambda i, c=c, r=r: (c, r * NB + i, 0))
        for c in range(NCORE) for r in range(R)
    ]
    cnt_specs = [
        pl.BlockSpec((1, BM, 1), lambda i, c=c, r=r: (c, r * NB + i, 0))
        for c in range(NCORE) for r in range(R)
    ]
    update_call = pl.pallas_call(
        _update_body,
        grid=(NB,),
        in_specs=[
            pl.BlockSpec((BM, D), lambda i: (i, 0)),
            pl.BlockSpec((D, D), lambda i: (0, 0)),
            pl.BlockSpec((1, D), lambda i: (0, 0)),
            pl.BlockSpec((NCORE, R, HD, D), lambda i: (0, 0, 0, 0)),
        ] + acc_specs + cnt_specs,
        out_specs=pl.BlockSpec((BM, D), lambda i: (i, 0)),
        out_shape=jax.ShapeDtypeStruct((N, D), F32),
    )

    xcur = x
    for l in range(L):
        xr = xcur.reshape(2 * N, HD)
        acc = layer_scatter(xr, ew2, gids, keys)
        xcur = update_call(xcur, roots[l], biases[l].reshape(1, D), wcat[l],
                           acc, acc, acc, acc, acc, acc,
                           cnt3, cnt3, cnt3, cnt3, cnt3, cnt3)
    return xcur
